# baseline (speedup 1.0000x reference)
import functools

import jax
import jax.numpy as jnp
from jax import lax
from jax.experimental import pallas as pl
from jax.experimental.pallas import tpu as pltpu
from jax.experimental.pallas import tpu_sc as plsc

NC = 2
NS = 16
L = 16
NW = NC * NS

EMB_DIM = 64
CHUNK = 128
NBUF = 4


def _make_sc_scores(total: int):
    per_w = total // NW
    assert per_w * NW == total and per_w % CHUNK == 0
    nchunk = per_w // CHUNK
    assert nchunk % NBUF == 0
    mesh = plsc.VectorSubcoreMesh(core_axis_name="c", subcore_axis_name="s")

    row_bufs = [pltpu.VMEM((CHUNK, EMB_DIM), jnp.float32) for _ in range(2 * NBUF)]
    sem_list = [pltpu.SemaphoreType.DMA for _ in range(2 * NBUF)]

    @functools.partial(
        pl.kernel,
        mesh=mesh,
        out_type=jax.ShapeDtypeStruct((total,), jnp.float32),
        compiler_params=pltpu.CompilerParams(
            needs_layout_passes=False, use_tc_tiling_on_sc=False),
        scratch_types=[
            pltpu.VMEM((nchunk, 2, CHUNK), jnp.int32),
            pltpu.VMEM((per_w,), jnp.float32),
            *row_bufs,
            *sem_list,
        ],
    )
    def sc_scores(emb_hbm, idx_hbm, out_hbm, idxv, outv, *rest):
        rows_v = rest[: 2 * NBUF]
        sems = rest[2 * NBUF:]
        wid = lax.axis_index("s") * NC + lax.axis_index("c")
        lane = lax.iota(jnp.int32, L)

        pltpu.sync_copy(idx_hbm.at[wid], idxv)

        def issue(g, b):
            cu = pltpu.async_copy(emb_hbm.at[idxv.at[g, 0]], rows_v[2 * b], sems[2 * b])
            cv = pltpu.async_copy(emb_hbm.at[idxv.at[g, 1]], rows_v[2 * b + 1], sems[2 * b + 1])
            return cu, cv

        def compute(g, b):
            urows = rows_v[2 * b]
            vrows = rows_v[2 * b + 1]

            def group_body(gr, carry):
                rows = gr * L + lane
                accs = [jnp.zeros((L,), jnp.float32) for _ in range(4)]
                for d in range(EMB_DIM):
                    col = (lane + d) & (EMB_DIM - 1)
                    uu = plsc.load_gather(urows, [rows, col])
                    vv = plsc.load_gather(vrows, [rows, col])
                    accs[d % 4] = accs[d % 4] + uu * vv
                acc = (accs[0] + accs[1]) + (accs[2] + accs[3])
                outv[pl.ds(g * CHUNK + gr * L, L)] = acc
                return carry

            lax.fori_loop(0, CHUNK // L, group_body, 0, unroll=False)

        def wait(b):
            pltpu.make_async_copy(emb_hbm.at[idxv.at[0, 0]], rows_v[2 * b], sems[2 * b]).wait()
            pltpu.make_async_copy(emb_hbm.at[idxv.at[0, 1]], rows_v[2 * b + 1], sems[2 * b + 1]).wait()

        for b in range(NBUF):
            issue(b, b)

        def outer_body(o, carry):
            for b in range(NBUF):
                g = o * NBUF + b
                wait(b)
                compute(g, b)
                issue(g + NBUF, b)
            return carry

        lax.fori_loop(0, nchunk // NBUF - 1, outer_body, 0, unroll=False)

        for b in range(NBUF):
            g = nchunk - NBUF + b
            wait(b)
            compute(g, b)

        pltpu.sync_copy(outv, out_hbm.at[pl.ds(wid * per_w, per_w)])

    return sc_scores


def _tc_loss_body(pos_ref, neg_ref, out_ref):
    pos = pos_ref[...]
    neg = neg_ref[...]
    ls_pos = jax.nn.log_sigmoid(pos)
    ls_neg = jax.nn.log_sigmoid(-neg)
    out_ref[...] = -(ls_pos + jnp.sum(ls_neg, axis=1, keepdims=True))


def kernel(emb, pos_u, pos_v, neg_u, neg_v):
    b = pos_u.shape[0]
    k = neg_u.shape[1]
    total = b * (1 + k)
    per_w = total // NW
    nchunk = per_w // CHUNK
    u_all = jnp.concatenate([pos_u.astype(jnp.int32), neg_u.reshape(-1).astype(jnp.int32)])
    v_all = jnp.concatenate([pos_v.astype(jnp.int32), neg_v.reshape(-1).astype(jnp.int32)])
    idx_slab = jnp.stack(
        [u_all.reshape(NW, nchunk, CHUNK), v_all.reshape(NW, nchunk, CHUNK)], axis=2)
    scores = _make_sc_scores(total)(emb, idx_slab)
    pos_s = scores[:b].reshape(b, 1)
    neg_s = scores[b:].reshape(b, k)
    loss2d = pl.pallas_call(
        _tc_loss_body,
        out_shape=jax.ShapeDtypeStruct((b, 1), jnp.float32),
    )(pos_s, neg_s)
    return loss2d.reshape(b)


# speedup vs baseline: 1.1394x; 1.1394x over previous
import functools

import jax
import jax.numpy as jnp
from jax import lax
from jax.experimental import pallas as pl
from jax.experimental.pallas import tpu as pltpu
from jax.experimental.pallas import tpu_sc as plsc

NC = 2
NS = 16
L = 16
NW = NC * NS

EMB_DIM = 64
CHUNK = 128
NBUF = 4


def _make_sc_scores(total: int):
    per_w = total // NW
    assert per_w * NW == total and per_w % CHUNK == 0
    nchunk = per_w // CHUNK
    assert nchunk % NBUF == 0
    mesh = plsc.VectorSubcoreMesh(core_axis_name="c", subcore_axis_name="s")

    row_bufs = [pltpu.VMEM((CHUNK, EMB_DIM), jnp.float32) for _ in range(2 * NBUF)]
    sem_list = [pltpu.SemaphoreType.DMA for _ in range(2 * NBUF)]

    @functools.partial(
        pl.kernel,
        mesh=mesh,
        out_type=jax.ShapeDtypeStruct((total,), jnp.float32),
        compiler_params=pltpu.CompilerParams(
            needs_layout_passes=False, use_tc_tiling_on_sc=False),
        scratch_types=[
            pltpu.VMEM((nchunk, 2, CHUNK), jnp.int32),
            pltpu.VMEM((per_w,), jnp.float32),
            *row_bufs,
            *sem_list,
        ],
    )
    def sc_scores(emb_hbm, idx_hbm, out_hbm, idxv, outv, *rest):
        rows_v = rest[: 2 * NBUF]
        sems = rest[2 * NBUF:]
        wid = lax.axis_index("s") * NC + lax.axis_index("c")
        lane = lax.iota(jnp.int32, L)

        pltpu.sync_copy(idx_hbm.at[wid], idxv)

        def issue(g, b):
            cu = pltpu.async_copy(emb_hbm.at[idxv.at[g, 0]], rows_v[2 * b], sems[2 * b])
            cv = pltpu.async_copy(emb_hbm.at[idxv.at[g, 1]], rows_v[2 * b + 1], sems[2 * b + 1])
            return cu, cv

        def compute(g, b):
            urows = rows_v[2 * b]
            vrows = rows_v[2 * b + 1]

            def group_body(gr, carry):
                rows = gr * L + lane
                acc0 = jnp.zeros((L,), jnp.float32)
                acc1 = jnp.zeros((L,), jnp.float32)
                for d in range(EMB_DIM):
                    col = lane ^ d
                    uu = plsc.load_gather(urows, [rows, col])
                    vv = plsc.load_gather(vrows, [rows, col])
                    if d % 2 == 0:
                        acc0 = acc0 + uu * vv
                    else:
                        acc1 = acc1 + uu * vv
                outv[pl.ds(g * CHUNK + gr * L, L)] = acc0 + acc1
                return carry

            lax.fori_loop(0, CHUNK // L, group_body, 0, unroll=False)

        def wait(b):
            pltpu.make_async_copy(emb_hbm.at[idxv.at[0, 0]], rows_v[2 * b], sems[2 * b]).wait()
            pltpu.make_async_copy(emb_hbm.at[idxv.at[0, 1]], rows_v[2 * b + 1], sems[2 * b + 1]).wait()

        for b in range(NBUF):
            issue(b, b)

        def outer_body(o, carry):
            for b in range(NBUF):
                g = o * NBUF + b
                wait(b)
                compute(g, b)
                issue(g + NBUF, b)
            return carry

        lax.fori_loop(0, nchunk // NBUF - 1, outer_body, 0, unroll=False)

        for b in range(NBUF):
            g = nchunk - NBUF + b
            wait(b)
            compute(g, b)

        pltpu.sync_copy(outv, out_hbm.at[pl.ds(wid * per_w, per_w)])

    return sc_scores


def _tc_loss_body(pos_ref, neg_ref, out_ref):
    pos = pos_ref[...]
    neg = neg_ref[...]
    ls_pos = jax.nn.log_sigmoid(pos)
    ls_neg = jax.nn.log_sigmoid(-neg)
    out_ref[...] = -(ls_pos + jnp.sum(ls_neg, axis=0, keepdims=True))


def kernel(emb, pos_u, pos_v, neg_u, neg_v):
    b = pos_u.shape[0]
    k = neg_u.shape[1]
    total = b * (1 + k)
    per_w = total // NW
    nchunk = per_w // CHUNK
    u_all = jnp.concatenate([pos_u.astype(jnp.int32), neg_u.T.reshape(-1).astype(jnp.int32)])
    v_all = jnp.concatenate([pos_v.astype(jnp.int32), neg_v.T.reshape(-1).astype(jnp.int32)])
    idx_slab = jnp.stack(
        [u_all.reshape(NW, nchunk, CHUNK), v_all.reshape(NW, nchunk, CHUNK)], axis=2)
    scores = _make_sc_scores(total)(emb, idx_slab)
    pos_s = scores[:b].reshape(1, b)
    neg_s = scores[b:].reshape(k, b)
    loss2d = pl.pallas_call(
        _tc_loss_body,
        out_shape=jax.ShapeDtypeStruct((1, b), jnp.float32),
    )(pos_s, neg_s)
    return loss2d.reshape(b)
